# baseline (speedup 1.0000x reference)
import jax
import jax.numpy as jnp
from jax.experimental import pallas as pl

N_TOK = 32768
EMB = 64
USER = 16
EXPERTS = 64

BLK = 2048


def _gate_kernel(h_ref, u_ref, wgt_ref, bg_ref, wbt_ref, bb_ref, wlt_ref,
                 bl_ref, triu_ref, out_ref):
    h = h_ref[...]
    u = u_ref[...]
    gamma = jnp.dot(u, wgt_ref[...], preferred_element_type=jnp.float32)
    beta = jnp.dot(u, wbt_ref[...], preferred_element_type=jnp.float32)
    h_t = h * (1.0 + gamma + bg_ref[...]) + (beta + bb_ref[...])
    logits = jnp.dot(h_t, wlt_ref[...],
                     preferred_element_type=jnp.float32) + bl_ref[...]

    triu = triu_ref[...]
    m1 = jnp.max(logits, axis=1, keepdims=True)
    eq1 = logits == m1
    eq1f = jnp.where(eq1, 1.0, 0.0)
    q1 = jnp.dot(eq1f, triu, preferred_element_type=jnp.float32)
    first1f = jnp.where(eq1 & (q1 == 1.0), 1.0, 0.0)
    second1f = jnp.where(eq1 & (q1 == 2.0), 1.0, 0.0)
    dup1f = jnp.where(q1[:, EXPERTS - 1:] >= 2.0, 1.0, 0.0)

    strict = jnp.where(eq1, -jnp.inf, logits)
    m2s = jnp.max(strict, axis=1, keepdims=True)
    eq2 = strict == m2s
    eq2f = jnp.where(eq2, 1.0, 0.0)
    q2 = jnp.dot(eq2f, triu, preferred_element_type=jnp.float32)
    first2f = jnp.where(eq2 & (q2 == 1.0), 1.0, 0.0)

    maskf = first1f + dup1f * second1f + (1.0 - dup1f) * first2f
    m2s_fin = jnp.maximum(m2s, -3.0e38)
    m2 = dup1f * m1 + (1.0 - dup1f) * m2s_fin
    scale = 1.0 / (1.0 + jnp.exp(m2 - m1))
    out_ref[...] = maskf * (jnp.exp(logits - m1) * scale)


@jax.jit
def _run(h, u, wgt, bg2, wbt, bb2, wlt, bl2, triu):
    grid = (N_TOK // BLK,)
    tok_spec = lambda width: pl.BlockSpec((BLK, width), lambda i: (i, 0))
    full = lambda a: pl.BlockSpec(a.shape, lambda i: (0,) * a.ndim)
    return pl.pallas_call(
        _gate_kernel,
        grid=grid,
        in_specs=[
            tok_spec(EMB),
            tok_spec(USER),
            full(wgt), full(bg2), full(wbt), full(bb2), full(wlt), full(bl2),
            full(triu),
        ],
        out_specs=tok_spec(EXPERTS),
        out_shape=jax.ShapeDtypeStruct((N_TOK, EXPERTS), jnp.float32),
    )(h, u, wgt, bg2, wbt, bb2, wlt, bl2, triu)


def kernel(h, u, Wg, bg, Wb, bb, Wl, bl):
    triu = jnp.triu(jnp.ones((EXPERTS, EXPERTS), jnp.float32))
    return _run(h, u, Wg.T, bg.reshape(1, EMB), Wb.T, bb.reshape(1, EMB),
                Wl.T, bl.reshape(1, EXPERTS), triu)


# speedup vs baseline: 1.4718x; 1.4718x over previous
import jax
import jax.numpy as jnp
from jax.experimental import pallas as pl

N_TOK = 32768
EMB = 64
USER = 16
EXPERTS = 64

BLK = 2048


def _gate_kernel(h_ref, u_ref, wg_ref, bg_ref, wb_ref, bb_ref, wl_ref,
                 blt_ref, out_ref):
    h = h_ref[...]
    u = u_ref[...]
    dn = (((1,), (1,)), ((), ()))
    gamma = jax.lax.dot_general(u, wg_ref[...], dn,
                                preferred_element_type=jnp.float32)
    beta = jax.lax.dot_general(u, wb_ref[...], dn,
                               preferred_element_type=jnp.float32)
    h_t = h * (1.0 + gamma + bg_ref[...]) + (beta + bb_ref[...])
    lt = jax.lax.dot_general(wl_ref[...], h_t, dn,
                             preferred_element_type=jnp.float32) + blt_ref[...]

    rows = jax.lax.broadcasted_iota(jnp.int32, lt.shape, 0)
    m1 = jnp.max(lt, axis=0, keepdims=True)
    i1 = jnp.min(jnp.where(lt == m1, rows, EXPERTS), axis=0, keepdims=True)
    sel1 = rows == i1
    rest = jnp.where(sel1, -jnp.inf, lt)
    m2 = jnp.max(rest, axis=0, keepdims=True)
    i2 = jnp.min(jnp.where(rest == m2, rows, EXPERTS), axis=0, keepdims=True)

    s1 = 1.0 / (1.0 + jnp.exp(m2 - m1))
    out_t = jnp.where(sel1, s1, 0.0) + jnp.where(rows == i2, 1.0 - s1, 0.0)
    out_ref[...] = out_t.T


@jax.jit
def _run(h, u, wg, bg2, wb, bb2, wl, blt):
    grid = (N_TOK // BLK,)
    tok_spec = lambda width: pl.BlockSpec((BLK, width), lambda i: (i, 0))
    full = lambda a: pl.BlockSpec(a.shape, lambda i: (0,) * a.ndim)
    return pl.pallas_call(
        _gate_kernel,
        grid=grid,
        in_specs=[
            tok_spec(EMB),
            tok_spec(USER),
            full(wg), full(bg2), full(wb), full(bb2), full(wl), full(blt),
        ],
        out_specs=tok_spec(EXPERTS),
        out_shape=jax.ShapeDtypeStruct((N_TOK, EXPERTS), jnp.float32),
    )(h, u, wg, bg2, wb, bb2, wl, blt)


def kernel(h, u, Wg, bg, Wb, bb, Wl, bl):
    return _run(h, u, Wg, bg.reshape(1, EMB), Wb, bb.reshape(1, EMB),
                Wl, bl.reshape(EXPERTS, 1))


# speedup vs baseline: 1.5871x; 1.0783x over previous
import jax
import jax.numpy as jnp
from jax.experimental import pallas as pl

N_TOK = 32768
EMB = 64
USER = 16
EXPERTS = 64

BLK = 4096


def _gate_kernel(h_ref, u_ref, wg_ref, bg_ref, wb_ref, bb_ref, wl_ref,
                 blt_ref, out_ref):
    h = h_ref[...]
    u = u_ref[...]
    dn = (((1,), (1,)), ((), ()))
    gamma = jax.lax.dot_general(u, wg_ref[...], dn,
                                preferred_element_type=jnp.float32)
    beta = jax.lax.dot_general(u, wb_ref[...], dn,
                               preferred_element_type=jnp.float32)
    h_t = h * (1.0 + gamma + bg_ref[...]) + (beta + bb_ref[...])
    lt = jax.lax.dot_general(wl_ref[...], h_t, dn,
                             preferred_element_type=jnp.float32) + blt_ref[...]

    rows = jax.lax.broadcasted_iota(jnp.int32, lt.shape, 0)
    m1 = jnp.max(lt, axis=0, keepdims=True)
    i1 = jnp.min(jnp.where(lt == m1, rows, EXPERTS), axis=0, keepdims=True)
    sel1 = rows == i1
    rest = jnp.where(sel1, -jnp.inf, lt)
    m2 = jnp.max(rest, axis=0, keepdims=True)
    i2 = jnp.min(jnp.where(rest == m2, rows, EXPERTS), axis=0, keepdims=True)

    s1 = 1.0 / (1.0 + jnp.exp(m2 - m1))
    out_t = jnp.where(sel1, s1, 0.0) + jnp.where(rows == i2, 1.0 - s1, 0.0)
    out_ref[...] = out_t.T


@jax.jit
def _run(h, u, wg, bg2, wb, bb2, wl, blt):
    grid = (N_TOK // BLK,)
    tok_spec = lambda width: pl.BlockSpec((BLK, width), lambda i: (i, 0))
    full = lambda a: pl.BlockSpec(a.shape, lambda i: (0,) * a.ndim)
    return pl.pallas_call(
        _gate_kernel,
        grid=grid,
        in_specs=[
            tok_spec(EMB),
            tok_spec(USER),
            full(wg), full(bg2), full(wb), full(bb2), full(wl), full(blt),
        ],
        out_specs=tok_spec(EXPERTS),
        out_shape=jax.ShapeDtypeStruct((N_TOK, EXPERTS), jnp.float32),
    )(h, u, wg, bg2, wb, bb2, wl, blt)


def kernel(h, u, Wg, bg, Wb, bb, Wl, bl):
    return _run(h, u, Wg, bg.reshape(1, EMB), Wb, bb.reshape(1, EMB),
                Wl, bl.reshape(EXPERTS, 1))


# speedup vs baseline: 1.6216x; 1.0217x over previous
import jax
import jax.numpy as jnp
from jax.experimental import pallas as pl

N_TOK = 32768
EMB = 64
USER = 16
EXPERTS = 64

BLK = 8192


def _gate_kernel(h_ref, u_ref, wg_ref, bg_ref, wb_ref, bb_ref, wl_ref,
                 blt_ref, out_ref):
    h = h_ref[...]
    u = u_ref[...]
    dn = (((1,), (1,)), ((), ()))
    gamma = jax.lax.dot_general(u, wg_ref[...], dn,
                                preferred_element_type=jnp.float32)
    beta = jax.lax.dot_general(u, wb_ref[...], dn,
                               preferred_element_type=jnp.float32)
    h_t = h * (1.0 + gamma + bg_ref[...]) + (beta + bb_ref[...])
    lt = jax.lax.dot_general(wl_ref[...], h_t, dn,
                             preferred_element_type=jnp.float32) + blt_ref[...]

    rows = jax.lax.broadcasted_iota(jnp.int32, lt.shape, 0)
    m1 = jnp.max(lt, axis=0, keepdims=True)
    i1 = jnp.min(jnp.where(lt == m1, rows, EXPERTS), axis=0, keepdims=True)
    sel1 = rows == i1
    rest = jnp.where(sel1, -jnp.inf, lt)
    m2 = jnp.max(rest, axis=0, keepdims=True)
    i2 = jnp.min(jnp.where(rest == m2, rows, EXPERTS), axis=0, keepdims=True)

    s1 = 1.0 / (1.0 + jnp.exp(m2 - m1))
    out_t = jnp.where(sel1, s1, 0.0) + jnp.where(rows == i2, 1.0 - s1, 0.0)
    out_ref[...] = out_t.T


@jax.jit
def _run(h, u, wg, bg2, wb, bb2, wl, blt):
    grid = (N_TOK // BLK,)
    tok_spec = lambda width: pl.BlockSpec((BLK, width), lambda i: (i, 0))
    full = lambda a: pl.BlockSpec(a.shape, lambda i: (0,) * a.ndim)
    return pl.pallas_call(
        _gate_kernel,
        grid=grid,
        in_specs=[
            tok_spec(EMB),
            tok_spec(USER),
            full(wg), full(bg2), full(wb), full(bb2), full(wl), full(blt),
        ],
        out_specs=tok_spec(EXPERTS),
        out_shape=jax.ShapeDtypeStruct((N_TOK, EXPERTS), jnp.float32),
    )(h, u, wg, bg2, wb, bb2, wl, blt)


def kernel(h, u, Wg, bg, Wb, bb, Wl, bl):
    return _run(h, u, Wg, bg.reshape(1, EMB), Wb, bb.reshape(1, EMB),
                Wl, bl.reshape(EXPERTS, 1))


# speedup vs baseline: 1.6377x; 1.0099x over previous
import jax
import jax.numpy as jnp
from jax.experimental import pallas as pl

N_TOK = 32768
EMB = 64
USER = 16
EXPERTS = 64

BLK = 8192


def _gate_kernel(h_ref, u_ref, wg_ref, bg_ref, wb_ref, bb_ref, wl_ref,
                 blt_ref, out_ref):
    h = h_ref[...]
    u = u_ref[...]
    dn = (((1,), (1,)), ((), ()))
    wgb = jnp.concatenate([wg_ref[...], wb_ref[...]], axis=0)
    bias = jnp.concatenate([1.0 + bg_ref[...], bb_ref[...]], axis=1)
    gb = jax.lax.dot_general(u, wgb, dn,
                             preferred_element_type=jnp.float32) + bias
    h_t = h * gb[:, :EMB] + gb[:, EMB:]
    lt = jax.lax.dot_general(wl_ref[...], h_t, dn,
                             preferred_element_type=jnp.float32) + blt_ref[...]

    rows = jax.lax.broadcasted_iota(jnp.int32, lt.shape, 0)
    m1 = jnp.max(lt, axis=0, keepdims=True)
    i1 = jnp.min(jnp.where(lt == m1, rows, EXPERTS), axis=0, keepdims=True)
    sel1 = rows == i1
    rest = jnp.where(sel1, -jnp.inf, lt)
    m2 = jnp.max(rest, axis=0, keepdims=True)
    i2 = jnp.min(jnp.where(rest == m2, rows, EXPERTS), axis=0, keepdims=True)

    s1 = 1.0 / (1.0 + jnp.exp(m2 - m1))
    out_t = jnp.where(sel1, s1, 0.0) + jnp.where(rows == i2, 1.0 - s1, 0.0)
    out_ref[...] = out_t.T


@jax.jit
def _run(h, u, wg, bg2, wb, bb2, wl, blt):
    grid = (N_TOK // BLK,)
    tok_spec = lambda width: pl.BlockSpec((BLK, width), lambda i: (i, 0))
    full = lambda a: pl.BlockSpec(a.shape, lambda i: (0,) * a.ndim)
    return pl.pallas_call(
        _gate_kernel,
        grid=grid,
        in_specs=[
            tok_spec(EMB),
            tok_spec(USER),
            full(wg), full(bg2), full(wb), full(bb2), full(wl), full(blt),
        ],
        out_specs=tok_spec(EXPERTS),
        out_shape=jax.ShapeDtypeStruct((N_TOK, EXPERTS), jnp.float32),
    )(h, u, wg, bg2, wb, bb2, wl, blt)


def kernel(h, u, Wg, bg, Wb, bb, Wl, bl):
    return _run(h, u, Wg, bg.reshape(1, EMB), Wb, bb.reshape(1, EMB),
                Wl, bl.reshape(EXPERTS, 1))
